# baseline (speedup 1.0000x reference)
import functools
import jax
import jax.numpy as jnp
from jax import lax
from jax.experimental import pallas as pl
from jax.experimental.pallas import tpu as pltpu
from jax.experimental.pallas import tpu_sc as plsc

N = 10000
E = 320000
D = 128
NC = 2
NS = 16
NW = NC * NS
EPW = E // NW
CH = 128
NFULL = EPW // CH
TAIL = EPW - NFULL * CH
RPS = 624
RTAIL = N - NS * RPS

_mesh = plsc.VectorSubcoreMesh(core_axis_name="c", subcore_axis_name="s")


@functools.partial(
    pl.kernel,
    out_type=jax.ShapeDtypeStruct((NC * N,), jnp.float32),
    mesh=_mesh,
    scratch_types=[
        pltpu.VMEM((CH,), jnp.int32),
        pltpu.VMEM((CH,), jnp.float32),
        pltpu.VMEM((TAIL,), jnp.int32),
        pltpu.VMEM((TAIL,), jnp.float32),
        pltpu.VMEM((RPS,), jnp.float32),
        pltpu.VMEM_SHARED((N,), jnp.float32),
    ],
)
def _sc_degree(col_hbm, deg_hbm, cidx, ones_v, cidx_t, ones_t, zb, acc):
    c = lax.axis_index("c")
    s = lax.axis_index("s")
    wid = s * NC + c

    for i in range(RPS // 16):
        zb[pl.ds(i * 16, 16)] = jnp.zeros((16,), jnp.float32)
    for i in range(CH // 16):
        ones_v[pl.ds(i * 16, 16)] = jnp.ones((16,), jnp.float32)
    for i in range(TAIL // 16):
        ones_t[pl.ds(i * 16, 16)] = jnp.ones((16,), jnp.float32)

    pltpu.sync_copy(zb, acc.at[pl.ds(s * RPS, RPS)])

    @pl.when(s == 0)
    def _():
        pltpu.sync_copy(zb.at[pl.ds(0, RTAIL)], acc.at[pl.ds(NS * RPS, RTAIL)])

    plsc.subcore_barrier()

    base = wid * EPW

    @pl.loop(0, NFULL)
    def _(i):
        pltpu.sync_copy(col_hbm.at[pl.ds(base + i * CH, CH)], cidx)
        pltpu.sync_copy(ones_v, acc.at[cidx], add=True)

    pltpu.sync_copy(col_hbm.at[pl.ds(base + NFULL * CH, TAIL)], cidx_t)
    pltpu.sync_copy(ones_t, acc.at[cidx_t], add=True)

    plsc.subcore_barrier()
    pltpu.sync_copy(acc.at[pl.ds(s * RPS, RPS)], zb)
    pltpu.sync_copy(zb, deg_hbm.at[pl.ds(c * N + s * RPS, RPS)])

    @pl.when(s == 0)
    def _():
        pltpu.sync_copy(acc.at[pl.ds(NS * RPS, RTAIL)], ones_t)
        pltpu.sync_copy(ones_t, deg_hbm.at[pl.ds(c * N + NS * RPS, RTAIL)])


@functools.partial(
    pl.kernel,
    out_type=jax.ShapeDtypeStruct((NC, N, D), jnp.float32),
    mesh=_mesh,
    scratch_types=[
        pltpu.VMEM((CH,), jnp.int32),
        pltpu.VMEM((CH,), jnp.int32),
        pltpu.VMEM((CH, D), jnp.float32),
        pltpu.VMEM((TAIL,), jnp.int32),
        pltpu.VMEM((TAIL,), jnp.int32),
        pltpu.VMEM((TAIL, D), jnp.float32),
        pltpu.VMEM_SHARED((N, D), jnp.float32),
        pltpu.SemaphoreType.DMA,
    ],
)
def _sc_scatter(row_hbm, col_hbm, h_hbm, zeros_hbm, out_hbm, ridx, cidx, rows,
                ridx_t, cidx_t, rows_t, acc, sem):
    c = lax.axis_index("c")
    s = lax.axis_index("s")
    wid = s * NC + c

    pltpu.sync_copy(zeros_hbm.at[pl.ds(s * RPS, RPS)],
                    acc.at[pl.ds(s * RPS, RPS)])

    @pl.when(s == 0)
    def _():
        pltpu.sync_copy(zeros_hbm.at[pl.ds(NS * RPS, RTAIL)],
                        acc.at[pl.ds(NS * RPS, RTAIL)])

    plsc.subcore_barrier()

    base = wid * EPW

    @pl.loop(0, NFULL)
    def _(i):
        pltpu.sync_copy(row_hbm.at[pl.ds(base + i * CH, CH)], ridx)
        pltpu.sync_copy(col_hbm.at[pl.ds(base + i * CH, CH)], cidx)
        pltpu.async_copy(h_hbm.at[ridx], rows, sem).wait()
        pltpu.sync_copy(rows, acc.at[cidx], add=True)

    pltpu.sync_copy(row_hbm.at[pl.ds(base + NFULL * CH, TAIL)], ridx_t)
    pltpu.sync_copy(col_hbm.at[pl.ds(base + NFULL * CH, TAIL)], cidx_t)
    pltpu.async_copy(h_hbm.at[ridx_t], rows_t, sem).wait()
    pltpu.sync_copy(rows_t, acc.at[cidx_t], add=True)

    plsc.subcore_barrier()
    pltpu.sync_copy(acc.at[pl.ds(s * RPS, RPS)],
                    out_hbm.at[c, pl.ds(s * RPS, RPS)])

    @pl.when(s == 0)
    def _():
        pltpu.sync_copy(acc.at[pl.ds(NS * RPS, RTAIL)],
                        out_hbm.at[c, pl.ds(NS * RPS, RTAIL)])


_RB = 1000


def _mm_body(x_ref, w_ref, d0_ref, d1_ref, h_ref):
    deg = d0_ref[0, 0, :] + d1_ref[0, 0, :] + 1.0
    dinv = lax.rsqrt(deg)
    h = jnp.dot(x_ref[...], w_ref[...], preferred_element_type=jnp.float32)
    h_ref[...] = h * dinv[:, None]


def _fin_body(p_ref, h_ref, d0_ref, d1_ref, b_ref, o_ref):
    deg = d0_ref[0, 0, :] + d1_ref[0, 0, :] + 1.0
    dinv = lax.rsqrt(deg)
    t = (p_ref[0] + p_ref[1] + h_ref[...]) * dinv[:, None] + b_ref[...]
    o_ref[...] = jnp.where(t >= 0, t, 0.2 * t)


def kernel(x, edge_index, W, b):
    row = edge_index[0]
    col = edge_index[1]
    zeros_nd = jnp.zeros((N, D), jnp.float32)

    degflat = _sc_degree(col)
    d0 = degflat[:N].reshape(N // _RB, 1, _RB)
    d1 = degflat[N:].reshape(N // _RB, 1, _RB)

    dspec = pl.BlockSpec((1, 1, _RB), lambda i: (i, 0, 0))
    h = pl.pallas_call(
        _mm_body,
        grid=(N // _RB,),
        in_specs=[
            pl.BlockSpec((_RB, D), lambda i: (i, 0)),
            pl.BlockSpec((D, D), lambda i: (0, 0)),
            dspec,
            dspec,
        ],
        out_specs=pl.BlockSpec((_RB, D), lambda i: (i, 0)),
        out_shape=jax.ShapeDtypeStruct((N, D), jnp.float32),
    )(x, W, d0, d1)

    partial = _sc_scatter(row, col, h, zeros_nd)

    out = pl.pallas_call(
        _fin_body,
        grid=(N // _RB,),
        in_specs=[
            pl.BlockSpec((NC, _RB, D), lambda i: (0, i, 0)),
            pl.BlockSpec((_RB, D), lambda i: (i, 0)),
            dspec,
            dspec,
            pl.BlockSpec((1, D), lambda i: (0, 0)),
        ],
        out_specs=pl.BlockSpec((_RB, D), lambda i: (i, 0)),
        out_shape=jax.ShapeDtypeStruct((N, D), jnp.float32),
    )(partial, h, d0, d1, b.reshape(1, D))

    return out


# speedup vs baseline: 1.0542x; 1.0542x over previous
import functools
import jax
import jax.numpy as jnp
from jax import lax
from jax.experimental import pallas as pl
from jax.experimental.pallas import tpu as pltpu
from jax.experimental.pallas import tpu_sc as plsc

N = 10000
E = 320000
D = 128
NC = 2
NS = 16
NW = NC * NS
EPW = E // NW
CH = 128
NFULL = EPW // CH
TAIL = EPW - NFULL * CH
RPS = 624
RTAIL = N - NS * RPS

_mesh = plsc.VectorSubcoreMesh(core_axis_name="c", subcore_axis_name="s")


@functools.partial(
    pl.kernel,
    out_type=jax.ShapeDtypeStruct((NC * N,), jnp.float32),
    mesh=_mesh,
    scratch_types=[
        pltpu.VMEM((CH,), jnp.int32),
        pltpu.VMEM((CH,), jnp.float32),
        pltpu.VMEM((TAIL,), jnp.int32),
        pltpu.VMEM((TAIL,), jnp.float32),
        pltpu.VMEM((RPS,), jnp.float32),
        pltpu.VMEM_SHARED((N,), jnp.float32),
    ],
)
def _sc_degree(col_hbm, deg_hbm, cidx, ones_v, cidx_t, ones_t, zb, acc):
    c = lax.axis_index("c")
    s = lax.axis_index("s")
    wid = s * NC + c

    for i in range(RPS // 16):
        zb[pl.ds(i * 16, 16)] = jnp.zeros((16,), jnp.float32)
    for i in range(CH // 16):
        ones_v[pl.ds(i * 16, 16)] = jnp.ones((16,), jnp.float32)
    for i in range(TAIL // 16):
        ones_t[pl.ds(i * 16, 16)] = jnp.ones((16,), jnp.float32)

    pltpu.sync_copy(zb, acc.at[pl.ds(s * RPS, RPS)])

    @pl.when(s == 0)
    def _():
        pltpu.sync_copy(zb.at[pl.ds(0, RTAIL)], acc.at[pl.ds(NS * RPS, RTAIL)])

    plsc.subcore_barrier()

    base = wid * EPW

    @pl.loop(0, NFULL)
    def _(i):
        pltpu.sync_copy(col_hbm.at[pl.ds(base + i * CH, CH)], cidx)
        pltpu.sync_copy(ones_v, acc.at[cidx], add=True)

    pltpu.sync_copy(col_hbm.at[pl.ds(base + NFULL * CH, TAIL)], cidx_t)
    pltpu.sync_copy(ones_t, acc.at[cidx_t], add=True)

    plsc.subcore_barrier()
    pltpu.sync_copy(acc.at[pl.ds(s * RPS, RPS)], zb)
    pltpu.sync_copy(zb, deg_hbm.at[pl.ds(c * N + s * RPS, RPS)])

    @pl.when(s == 0)
    def _():
        pltpu.sync_copy(acc.at[pl.ds(NS * RPS, RTAIL)], ones_t)
        pltpu.sync_copy(ones_t, deg_hbm.at[pl.ds(c * N + NS * RPS, RTAIL)])


@functools.partial(
    pl.kernel,
    out_type=jax.ShapeDtypeStruct((NC, N, D), jnp.float32),
    mesh=_mesh,
    scratch_types=[
        [pltpu.VMEM((CH,), jnp.int32)] * 2,
        [pltpu.VMEM((CH,), jnp.int32)] * 2,
        [pltpu.VMEM((CH, D), jnp.float32)] * 2,
        pltpu.VMEM((TAIL,), jnp.int32),
        pltpu.VMEM((TAIL,), jnp.int32),
        pltpu.VMEM((TAIL, D), jnp.float32),
        pltpu.VMEM_SHARED((N, D), jnp.float32),
        [pltpu.SemaphoreType.DMA] * 2,
    ],
)
def _sc_scatter(row_hbm, col_hbm, h_hbm, zeros_hbm, out_hbm, ridx, cidx, rows,
                ridx_t, cidx_t, rows_t, acc, sem):
    c = lax.axis_index("c")
    s = lax.axis_index("s")
    wid = s * NC + c

    pltpu.sync_copy(zeros_hbm.at[pl.ds(s * RPS, RPS)],
                    acc.at[pl.ds(s * RPS, RPS)])

    @pl.when(s == 0)
    def _():
        pltpu.sync_copy(zeros_hbm.at[pl.ds(NS * RPS, RTAIL)],
                        acc.at[pl.ds(NS * RPS, RTAIL)])

    plsc.subcore_barrier()

    base = wid * EPW

    @pl.loop(0, NFULL // 2)
    def _(k):
        i = 2 * k
        pltpu.sync_copy(row_hbm.at[pl.ds(base + i * CH, CH)], ridx[0])
        pltpu.sync_copy(col_hbm.at[pl.ds(base + i * CH, CH)], cidx[0])
        pltpu.sync_copy(row_hbm.at[pl.ds(base + (i + 1) * CH, CH)], ridx[1])
        pltpu.sync_copy(col_hbm.at[pl.ds(base + (i + 1) * CH, CH)], cidx[1])
        d0 = pltpu.async_copy(h_hbm.at[ridx[0]], rows[0], sem[0])
        d1 = pltpu.async_copy(h_hbm.at[ridx[1]], rows[1], sem[1])
        d0.wait()
        pltpu.sync_copy(rows[0], acc.at[cidx[0]], add=True)
        d1.wait()
        pltpu.sync_copy(rows[1], acc.at[cidx[1]], add=True)

    pltpu.sync_copy(row_hbm.at[pl.ds(base + NFULL * CH, TAIL)], ridx_t)
    pltpu.sync_copy(col_hbm.at[pl.ds(base + NFULL * CH, TAIL)], cidx_t)
    pltpu.async_copy(h_hbm.at[ridx_t], rows_t, sem[0]).wait()
    pltpu.sync_copy(rows_t, acc.at[cidx_t], add=True)

    plsc.subcore_barrier()
    pltpu.sync_copy(acc.at[pl.ds(s * RPS, RPS)],
                    out_hbm.at[c, pl.ds(s * RPS, RPS)])

    @pl.when(s == 0)
    def _():
        pltpu.sync_copy(acc.at[pl.ds(NS * RPS, RTAIL)],
                        out_hbm.at[c, pl.ds(NS * RPS, RTAIL)])


_RB = 1000


def _mm_body(x_ref, w_ref, d0_ref, d1_ref, h_ref):
    deg = d0_ref[0, 0, :] + d1_ref[0, 0, :] + 1.0
    dinv = lax.rsqrt(deg)
    h = jnp.dot(x_ref[...], w_ref[...], preferred_element_type=jnp.float32)
    h_ref[...] = h * dinv[:, None]


def _fin_body(p_ref, h_ref, d0_ref, d1_ref, b_ref, o_ref):
    deg = d0_ref[0, 0, :] + d1_ref[0, 0, :] + 1.0
    dinv = lax.rsqrt(deg)
    t = (p_ref[0] + p_ref[1] + h_ref[...]) * dinv[:, None] + b_ref[...]
    o_ref[...] = jnp.where(t >= 0, t, 0.2 * t)


def kernel(x, edge_index, W, b):
    row = edge_index[0]
    col = edge_index[1]
    zeros_nd = jnp.zeros((N, D), jnp.float32)

    degflat = _sc_degree(col)
    d0 = degflat[:N].reshape(N // _RB, 1, _RB)
    d1 = degflat[N:].reshape(N // _RB, 1, _RB)

    dspec = pl.BlockSpec((1, 1, _RB), lambda i: (i, 0, 0))
    h = pl.pallas_call(
        _mm_body,
        grid=(N // _RB,),
        in_specs=[
            pl.BlockSpec((_RB, D), lambda i: (i, 0)),
            pl.BlockSpec((D, D), lambda i: (0, 0)),
            dspec,
            dspec,
        ],
        out_specs=pl.BlockSpec((_RB, D), lambda i: (i, 0)),
        out_shape=jax.ShapeDtypeStruct((N, D), jnp.float32),
    )(x, W, d0, d1)

    partial = _sc_scatter(row, col, h, zeros_nd)

    out = pl.pallas_call(
        _fin_body,
        grid=(N // _RB,),
        in_specs=[
            pl.BlockSpec((NC, _RB, D), lambda i: (0, i, 0)),
            pl.BlockSpec((_RB, D), lambda i: (i, 0)),
            dspec,
            dspec,
            pl.BlockSpec((1, D), lambda i: (0, 0)),
        ],
        out_specs=pl.BlockSpec((_RB, D), lambda i: (i, 0)),
        out_shape=jax.ShapeDtypeStruct((N, D), jnp.float32),
    )(partial, h, d0, d1, b.reshape(1, D))

    return out
